# Initial kernel scaffold; baseline (speedup 1.0000x reference)
import jax
import jax.numpy as jnp
from jax.experimental import pallas as pl


def kernel(features_1, edge_index_1, features_2, edge_index_2, W1, b1, W2, b2, W3, b3, att_W, ntn_W, ntn_V, ntn_b, fc_W, fc_b, sc_W, sc_b):
    raise NotImplementedError("write your pallas kernel here")


# speedup vs baseline: 11.1582x; 11.1582x over previous
import functools

import jax
import jax.numpy as jnp
from jax import lax
from jax.experimental import pallas as pl
from jax.experimental.pallas import tpu as pltpu
from jax.experimental.pallas import tpu_sc as plsc

N = 10000
E = 320000
D = 128
F1, F2, F3 = 128, 64, 32
T = 16
BN = 16

NP = 10112
N2 = 2 * NP
NC = 2
NS = 16
EPT = E // NS
K = 128
CH = (EPT + K - 1) // K
EPTP = CH * K
RPT = NP // NS
WD = 128
FW = 128

_mesh = plsc.VectorSubcoreMesh(core_axis_name="c", subcore_axis_name="s")


@functools.partial(
    pl.kernel,
    out_type=jax.ShapeDtypeStruct((NC, NP, FW), jnp.float32),
    mesh=_mesh,
    scratch_types=[
        pltpu.VMEM((K,), jnp.int32),
        pltpu.VMEM((CH, K), jnp.int32),
        pltpu.VMEM((K, FW), jnp.float32),
        pltpu.VMEM_SHARED((NP, FW), jnp.float32),
        pltpu.SemaphoreType.DMA,
    ],
    name="edge_scatter",
)
def _edge_scatter(table, src, dst, zeros, out, src_v, dst_v, rows_v, acc, sem):
    c = lax.axis_index("c")
    s = lax.axis_index("s")
    pltpu.sync_copy(zeros.at[pl.ds(s * RPT, RPT)], acc.at[pl.ds(s * RPT, RPT)])
    pltpu.sync_copy(dst.at[c, s], dst_v)
    plsc.subcore_barrier()

    def body(j, carry):
        pltpu.sync_copy(src.at[c, s, j], src_v)
        pltpu.async_copy(table.at[src_v], rows_v, sem).wait()
        pltpu.sync_copy(rows_v, acc.at[dst_v.at[j]], add=True)
        return carry

    lax.fori_loop(0, CH, body, 0)
    plsc.subcore_barrier()
    pltpu.sync_copy(acc.at[pl.ds(s * RPT, RPT)], out.at[c, pl.ds(s * RPT, RPT)])


@functools.partial(
    pl.kernel,
    out_type=jax.ShapeDtypeStruct((NC, NP, WD), jnp.float32),
    mesh=_mesh,
    scratch_types=[
        pltpu.VMEM((CH, K), jnp.int32),
        pltpu.VMEM((K, WD), jnp.float32),
        pltpu.VMEM_SHARED((NP, WD), jnp.float32),
    ],
    name="deg_scatter",
)
def _deg_scatter(dst, ones, zeros, out, dst_v, ones_v, acc):
    c = lax.axis_index("c")
    s = lax.axis_index("s")
    pltpu.sync_copy(zeros.at[pl.ds(s * RPT, RPT)], acc.at[pl.ds(s * RPT, RPT)])
    pltpu.sync_copy(dst.at[c, s], dst_v)
    pltpu.sync_copy(ones, ones_v)
    plsc.subcore_barrier()

    def body(j, carry):
        pltpu.sync_copy(ones_v, acc.at[dst_v.at[j]], add=True)
        return carry

    lax.fori_loop(0, CH, body, 0)
    plsc.subcore_barrier()
    pltpu.sync_copy(acc.at[pl.ds(s * RPT, RPT)], out.at[c, pl.ds(s * RPT, RPT)])


def _first_body(deg_ref, x_ref, w_ref, dinv_ref, g_ref):
    dinv = lax.rsqrt(deg_ref[:, 0:1] + 1.0)
    dinv_ref[...] = dinv
    g_ref[...] = dinv * jnp.dot(x_ref[...], w_ref[...],
                                preferred_element_type=jnp.float32)


def _make_mid_body(fin, fout):
    def mid(agg_ref, g_ref, dinv_ref, b_ref, w_ref, gn_ref):
        dinv = dinv_ref[...]
        h = jnp.maximum(
            dinv * (agg_ref[:, :fin] + g_ref[:, :fin]) + b_ref[...], 0.0)
        gn = dinv * jnp.dot(h, w_ref[...], preferred_element_type=jnp.float32)
        gn_ref[...] = jnp.concatenate(
            [gn, jnp.zeros((N2, FW - fout), jnp.float32)], axis=1)
    return mid


def _head_body(agg_ref, g_ref, dinv_ref, b_ref, attw_ref, wa_ref, e1_ref,
               e2_ref, vt_ref, ntnb_ref, fcw_ref, fcb_ref, scw_ref, scb_ref,
               out_ref):
    f = dinv_ref[...] * (agg_ref[:, :F3] + g_ref[:, :F3]) + b_ref[...]
    fa = f[0:N]
    fb = f[NP:NP + N]
    attw = attw_ref[...]

    def att(fg):
        gc = jnp.tanh(jnp.mean(
            jnp.dot(fg, attw, preferred_element_type=jnp.float32),
            axis=0, keepdims=True))
        sg = jax.nn.sigmoid(jnp.sum(fg * gc, axis=1, keepdims=True))
        return jnp.sum(fg * sg, axis=0, keepdims=True)

    p1 = att(fa)
    p2 = att(fb)
    dot = functools.partial(jnp.dot, preferred_element_type=jnp.float32)
    q = dot(p1, wa_ref[...])
    p2e = dot(p2, e2_ref[...])
    scoring = dot(q * p2e, e1_ref[...])
    comb = jnp.concatenate([p1, p2], axis=1)
    block = dot(comb, vt_ref[...])
    sc = jnp.maximum(scoring + block + ntnb_ref[...], 0.0)
    s = jnp.maximum(dot(sc, fcw_ref[...]) + fcb_ref[...], 0.0)
    out_ref[...] = jax.nn.sigmoid(dot(s, scw_ref[...]) + scb_ref[...])


def _tc(body, out_shape, *args):
    return pl.pallas_call(body, out_shape=out_shape)(*args)


def _prep_edges(ei, off):
    src = ei[0].reshape(NS, EPT)
    dst = ei[1].reshape(NS, EPT)
    src = jnp.pad(src, ((0, 0), (0, EPTP - EPT))) + off
    dst = jnp.pad(dst, ((0, 0), (0, EPTP - EPT)), constant_values=N)
    return src.reshape(NS, CH, K), dst.reshape(NS, CH, K)


def kernel(features_1, edge_index_1, features_2, edge_index_2, W1, b1, W2, b2,
           W3, b3, att_W, ntn_W, ntn_V, ntn_b, fc_W, fc_b, sc_W, sc_b):
    f32 = jnp.float32
    zrows = jnp.zeros((NP - N, D), f32)
    x_both = jnp.concatenate([features_1, zrows, features_2, zrows], axis=0)

    s1, d1 = _prep_edges(edge_index_1, 0)
    s2, d2 = _prep_edges(edge_index_2, NP)
    src = jnp.stack([s1, s2])
    dst = jnp.stack([d1, d2])
    zt = jnp.zeros((NP, FW), f32)

    deg = _deg_scatter(dst, jnp.ones((K, WD), f32), jnp.zeros((NP, WD), f32))
    deg = deg.reshape(N2, WD)

    dinv, g1 = _tc(
        _first_body,
        (jax.ShapeDtypeStruct((N2, 1), f32), jax.ShapeDtypeStruct((N2, F1), f32)),
        deg, x_both, W1)

    a1 = _edge_scatter(g1, src, dst, zt)
    g2 = _tc(_make_mid_body(F1, F2), jax.ShapeDtypeStruct((N2, FW), f32),
             a1.reshape(N2, FW), g1, dinv, b1[None, :], W2)

    a2 = _edge_scatter(g2, src, dst, zt)
    g3 = _tc(_make_mid_body(F2, F3), jax.ShapeDtypeStruct((N2, FW), f32),
             a2.reshape(N2, FW), g2, dinv, b2[None, :], W3)

    a3 = _edge_scatter(g3, src, dst, zt)

    wa = ntn_W.reshape(F3, F3 * T)
    e1 = jnp.kron(jnp.ones((F3, 1), f32), jnp.eye(T, dtype=f32))
    e2 = jnp.kron(jnp.eye(F3, dtype=f32), jnp.ones((1, T), f32))
    out = _tc(_head_body, jax.ShapeDtypeStruct((1, 1), f32),
              a3.reshape(N2, FW), g3, dinv, b3[None, :], att_W, wa, e1, e2,
              ntn_V.T, ntn_b.T, fc_W, fc_b[None, :], sc_W, sc_b[None, :])
    return out
